# Initial kernel scaffold; baseline (speedup 1.0000x reference)
import jax
import jax.numpy as jnp
from jax.experimental import pallas as pl


def kernel(x, edge_index, edge_attr, en0_W1, en0_b1, en0_W2, en0_b2, root0, bias0, en1_W1, en1_b1, en1_W2, en1_b2, root1, bias1, cls_W, cls_b):
    raise NotImplementedError("write your pallas kernel here")


# speedup vs baseline: 2.4610x; 2.4610x over previous
import functools

import jax
import jax.numpy as jnp
from jax import lax
from jax.experimental import pallas as pl
from jax.experimental.pallas import tpu as pltpu
from jax.experimental.pallas import tpu_sc as plsc

_N = 10000
_E = 160000
_DIN = 128
_DH = 16
_DE = 16
_KD = 16
_TW = (_KD + 1) * _DH

_E8 = _E // 8

_NW = 32
_EPW = _E // _NW
_B = 40
_NB = _EPW // _B
_NSUB = 16
_NPAD = 10240
_RPS = _NPAD // _NSUB

_f32 = jnp.float32


def _elu(v):
    return jnp.where(v > 0, v, jnp.exp(jnp.minimum(v, 0.0)) - 1.0)


def _edgenet_body(ea_ref, k0_ref, b0_ref, k1_ref, b1_ref, h0_ref, h1_ref):
    ea = ea_ref[...]
    h0_ref[...] = _elu(jnp.dot(ea, k0_ref[...], preferred_element_type=_f32)
                       + b0_ref[...])
    h1_ref[...] = _elu(jnp.dot(ea, k1_ref[...], preferred_element_type=_f32)
                       + b1_ref[...])


def _node_tab_body(x_ref, m_ref, rt_ref, t_ref, r_ref):
    xb = x_ref[...]
    t_ref[...] = jnp.dot(xb, m_ref[...], preferred_element_type=_f32)
    r_ref[...] = jnp.dot(xb, rt_ref[...], preferred_element_type=_f32)


def _mid_body(sa_ref, sb_ref, ca_ref, cb_ref, r0_ref, b0_ref, m1_ref, rt1_ref,
              t1_ref, r1_ref):
    cnt = jnp.maximum(ca_ref[...] + cb_ref[...], 1.0)
    aggr = (sa_ref[...] + sb_ref[...]) / cnt
    h1 = _elu(aggr + r0_ref[...] + b0_ref[...])
    t1_ref[...] = jnp.dot(h1, m1_ref[...], preferred_element_type=_f32)
    r1_ref[...] = jnp.dot(h1, rt1_ref[...], preferred_element_type=_f32)


def _fin_body(sa_ref, sb_ref, ca_ref, cb_ref, r1_ref, b1_ref, w_ref, cb2_ref,
              o_ref):
    cnt = jnp.maximum(ca_ref[...] + cb_ref[...], 1.0)
    aggr = (sa_ref[...] + sb_ref[...]) / cnt
    h2 = _elu(aggr + r1_ref[...] + b1_ref[...])
    o_ref[...] = (jnp.dot(h2, w_ref[...], preferred_element_type=_f32)
                  + cb2_ref[...])


def _full_spec(shape):
    return pl.BlockSpec(shape, lambda i: (0,) * len(shape))


def _row_spec(bn, w):
    return pl.BlockSpec((bn, w), lambda i: (i, 0))


_BE = 2000
_BN = 2000

_edgenet_call = pl.pallas_call(
    _edgenet_body,
    grid=(_E8 // _BE,),
    in_specs=[_row_spec(_BE, 128), _full_spec((128, 128)), _full_spec((1, 128)),
              _full_spec((128, 128)), _full_spec((1, 128))],
    out_specs=[_row_spec(_BE, 128), _row_spec(_BE, 128)],
    out_shape=[jax.ShapeDtypeStruct((_E8, 128), _f32),
               jax.ShapeDtypeStruct((_E8, 128), _f32)],
)

_node_tab_call = pl.pallas_call(
    _node_tab_body,
    grid=(_N // _BN,),
    in_specs=[_row_spec(_BN, _DIN), _full_spec((_DIN, _TW)),
              _full_spec((_DIN, _DH))],
    out_specs=[_row_spec(_BN, _TW), _row_spec(_BN, _DH)],
    out_shape=[jax.ShapeDtypeStruct((_N, _TW), _f32),
               jax.ShapeDtypeStruct((_N, _DH), _f32)],
)

_mid_call = pl.pallas_call(
    _mid_body,
    grid=(_N // _BN,),
    in_specs=[_row_spec(_BN, _DH)] * 5 + [_full_spec((1, _DH)),
              _full_spec((_DH, _TW)), _full_spec((_DH, _DH))],
    out_specs=[_row_spec(_BN, _TW), _row_spec(_BN, _DH)],
    out_shape=[jax.ShapeDtypeStruct((_N, _TW), _f32),
               jax.ShapeDtypeStruct((_N, _DH), _f32)],
)

_fin_call = pl.pallas_call(
    _fin_body,
    grid=(_N // _BN,),
    in_specs=[_row_spec(_BN, _DH)] * 5 + [_full_spec((1, _DH)),
              _full_spec((_DH, 10)), _full_spec((1, 10))],
    out_specs=_row_spec(_BN, 10),
    out_shape=jax.ShapeDtypeStruct((_N, 10), _f32),
)


def _build_edge_kernel(with_count):
    mesh = plsc.VectorSubcoreMesh(core_axis_name="c", subcore_axis_name="s")

    def body(*refs):
        if with_count:
            (t_hbm, h_hbm, src_hbm, dst_hbm, s_out, c_out,
             src_v, dst_v, h_v, rows_v, msg_v, ones_v, zer_v,
             s_sh, c_sh, sem) = refs
        else:
            (t_hbm, h_hbm, src_hbm, dst_hbm, s_out,
             src_v, dst_v, h_v, rows_v, msg_v, zer_v,
             s_sh, sem) = refs
        c = lax.axis_index("c")
        s = lax.axis_index("s")
        wid = s * 2 + c

        def zrow(i, _):
            zer_v[i, :] = jnp.zeros((_DH,), _f32)
            return 0
        lax.fori_loop(0, _RPS, zrow, 0)
        if with_count:
            def orow(i, _):
                ones_v[i, :] = jnp.full((_DH,), 1.0, _f32)
                return 0
            lax.fori_loop(0, _B, orow, 0)

        row0 = s * _RPS
        pltpu.sync_copy(zer_v, s_sh.at[pl.ds(row0, _RPS)])
        if with_count:
            pltpu.sync_copy(zer_v, c_sh.at[pl.ds(row0, _RPS)])
        plsc.subcore_barrier()

        ebase = wid * _EPW

        def batch(bi, _):
            e0 = ebase + bi * _B
            pltpu.sync_copy(src_hbm.at[pl.ds(e0, _B)], src_v)
            pltpu.sync_copy(dst_hbm.at[pl.ds(e0, _B)], dst_v)
            pltpu.sync_copy(h_hbm.at[pl.ds(e0, _B)], h_v)
            pltpu.async_copy(t_hbm.at[src_v], rows_v, sem).wait()

            def edge(e, _):
                hrow = h_v[e, :]
                acc = rows_v[e, pl.ds(_KD * _DH, _DH)]
                for k in range(_KD):
                    acc = acc + hrow[k] * rows_v[e, pl.ds(k * _DH, _DH)]
                msg_v[e, :] = acc
                return 0
            lax.fori_loop(0, _B, edge, 0)

            pltpu.sync_copy(msg_v, s_sh.at[dst_v], add=True)
            if with_count:
                pltpu.sync_copy(ones_v, c_sh.at[dst_v], add=True)
            return 0
        lax.fori_loop(0, _NB, batch, 0)

        plsc.subcore_barrier()
        pltpu.sync_copy(s_sh.at[pl.ds(row0, _RPS)],
                        s_out.at[c, pl.ds(row0, _RPS)])
        if with_count:
            pltpu.sync_copy(c_sh.at[pl.ds(row0, _RPS)],
                            c_out.at[c, pl.ds(row0, _RPS)])

    out_type = [jax.ShapeDtypeStruct((2, _NPAD, _DH), _f32)]
    scratch = [
        pltpu.VMEM((_B,), jnp.int32),
        pltpu.VMEM((_B,), jnp.int32),
        pltpu.VMEM((_B, _DH), _f32),
        pltpu.VMEM((_B, _TW), _f32),
        pltpu.VMEM((_B, _DH), _f32),
    ]
    if with_count:
        out_type.append(jax.ShapeDtypeStruct((2, _NPAD, _DH), _f32))
        scratch.append(pltpu.VMEM((_B, _DH), _f32))
    scratch.append(pltpu.VMEM((_RPS, _DH), _f32))
    scratch.append(pltpu.VMEM_SHARED((_NPAD, _DH), _f32))
    if with_count:
        scratch.append(pltpu.VMEM_SHARED((_NPAD, _DH), _f32))
    scratch.append(pltpu.SemaphoreType.DMA)

    return pl.kernel(body, out_type=out_type, mesh=mesh,
                     scratch_types=scratch,
                     compiler_params=pltpu.CompilerParams(
                         use_tc_tiling_on_sc=False))


_edge_call_cnt = _build_edge_kernel(with_count=True)
_edge_call_nocnt = _build_edge_kernel(with_count=False)


def kernel(x, edge_index, edge_attr, en0_W1, en0_b1, en0_W2, en0_b2, root0,
           bias0, en1_W1, en1_b1, en1_W2, en1_b2, root1, bias1, cls_W, cls_b):
    src = edge_index[0]
    dst = edge_index[1]

    eye8 = jnp.eye(8, dtype=_f32)
    k0 = jnp.kron(eye8, en0_W1)
    k1 = jnp.kron(eye8, en1_W1)
    tb0 = jnp.tile(en0_b1, 8)[None, :]
    tb1 = jnp.tile(en1_b1, 8)[None, :]
    ea8 = edge_attr.reshape(_E8, 128)
    he0_r, he1_r = _edgenet_call(ea8, k0, tb0, k1, tb1)
    he0 = he0_r.reshape(_E, _DE)
    he1 = he1_r.reshape(_E, _DE)

    m0 = en0_W2.reshape(_DE, _DIN, _DH).transpose(1, 0, 2).reshape(_DIN, _KD * _DH)
    m0aug = jnp.concatenate([m0, en0_b2.reshape(_DIN, _DH)], axis=1)
    t0, r0 = _node_tab_call(x, m0aug, root0)

    s0, c0 = _edge_call_cnt(t0, he0, src, dst)

    m1 = en1_W2.reshape(_DE, _DH, _DH).transpose(1, 0, 2).reshape(_DH, _KD * _DH)
    m1aug = jnp.concatenate([m1, en1_b2.reshape(_DH, _DH)], axis=1)
    t1, r1 = _mid_call(s0[0], s0[1], c0[0], c0[1], r0, bias0[None, :],
                       m1aug, root1)

    (s1,) = _edge_call_nocnt(t1, he1, src, dst)

    return _fin_call(s1[0], s1[1], c0[0], c0[1], r1, bias1[None, :],
                     cls_W, cls_b[None, :])


# speedup vs baseline: 4.6674x; 1.8966x over previous
import functools

import jax
import jax.numpy as jnp
from jax import lax
from jax.experimental import pallas as pl
from jax.experimental.pallas import tpu as pltpu
from jax.experimental.pallas import tpu_sc as plsc

_N = 10000
_E = 160000
_DIN = 128
_DH = 16
_DE = 16
_KD = 16
_TW = (_KD + 2) * _DH
_BF_PERM = tuple(
    (2 * p + half) * _DH + i
    for p in range(_TW // 32) for i in range(_DH) for half in (0, 1)
)

_E8 = _E // 8

_NW = 32
_EPW = _E // _NW
_B = 40
_CPG = 5
_G = 200
_NG = _EPW // _G
_NSUB = 16
_NPAD = 10240
_RPS = _NPAD // _NSUB

_f32 = jnp.float32


def _elu(v):
    return jnp.where(v > 0, v, jnp.exp(jnp.minimum(v, 0.0)) - 1.0)


def _edgenet_body(ea_ref, k0_ref, b0_ref, k1_ref, b1_ref, h0_ref, h1_ref):
    ea = ea_ref[...]
    h0_ref[...] = _elu(jnp.dot(ea, k0_ref[...], preferred_element_type=_f32)
                       + b0_ref[...])
    h1_ref[...] = _elu(jnp.dot(ea, k1_ref[...], preferred_element_type=_f32)
                       + b1_ref[...])


def _node_tab_body(x_ref, m_ref, rt_ref, t_ref, r_ref):
    xb = x_ref[...]
    t_ref[...] = jnp.dot(xb, m_ref[...],
                         preferred_element_type=_f32).astype(jnp.bfloat16)
    r_ref[...] = jnp.dot(xb, rt_ref[...], preferred_element_type=_f32)


def _mid_body(sa_ref, sb_ref, ca_ref, cb_ref, r0_ref, b0_ref, m1_ref, rt1_ref,
              t1_ref, r1_ref):
    cnt = jnp.maximum(ca_ref[...] + cb_ref[...], 1.0)
    aggr = (sa_ref[...] + sb_ref[...]) / cnt
    h1 = _elu(aggr + r0_ref[...] + b0_ref[...])
    t1_ref[...] = jnp.dot(h1, m1_ref[...],
                          preferred_element_type=_f32).astype(jnp.bfloat16)
    r1_ref[...] = jnp.dot(h1, rt1_ref[...], preferred_element_type=_f32)


def _fin_body(sa_ref, sb_ref, ca_ref, cb_ref, r1_ref, b1_ref, w_ref, cb2_ref,
              o_ref):
    cnt = jnp.maximum(ca_ref[...] + cb_ref[...], 1.0)
    aggr = (sa_ref[...] + sb_ref[...]) / cnt
    h2 = _elu(aggr + r1_ref[...] + b1_ref[...])
    o_ref[...] = (jnp.dot(h2, w_ref[...], preferred_element_type=_f32)
                  + cb2_ref[...])


def _full_spec(shape):
    return pl.BlockSpec(shape, lambda i: (0,) * len(shape))


def _row_spec(bn, w):
    return pl.BlockSpec((bn, w), lambda i: (i, 0))


_BE = 2000
_BN = 2000

_edgenet_call = pl.pallas_call(
    _edgenet_body,
    grid=(_E8 // _BE,),
    in_specs=[_row_spec(_BE, 128), _full_spec((128, 128)), _full_spec((1, 128)),
              _full_spec((128, 128)), _full_spec((1, 128))],
    out_specs=[_row_spec(_BE, 128), _row_spec(_BE, 128)],
    out_shape=[jax.ShapeDtypeStruct((_E8, 128), _f32),
               jax.ShapeDtypeStruct((_E8, 128), _f32)],
)

_node_tab_call = pl.pallas_call(
    _node_tab_body,
    grid=(_N // _BN,),
    in_specs=[_row_spec(_BN, _DIN), _full_spec((_DIN, _TW)),
              _full_spec((_DIN, _DH))],
    out_specs=[_row_spec(_BN, _TW), _row_spec(_BN, _DH)],
    out_shape=[jax.ShapeDtypeStruct((_N, _TW), jnp.bfloat16),
               jax.ShapeDtypeStruct((_N, _DH), _f32)],
)

_mid_call = pl.pallas_call(
    _mid_body,
    grid=(_N // _BN,),
    in_specs=[_row_spec(_BN, _DH)] * 5 + [_full_spec((1, _DH)),
              _full_spec((_DH, _TW)), _full_spec((_DH, _DH))],
    out_specs=[_row_spec(_BN, _TW), _row_spec(_BN, _DH)],
    out_shape=[jax.ShapeDtypeStruct((_N, _TW), jnp.bfloat16),
               jax.ShapeDtypeStruct((_N, _DH), _f32)],
)

_fin_call = pl.pallas_call(
    _fin_body,
    grid=(_N // _BN,),
    in_specs=[_row_spec(_BN, _DH)] * 5 + [_full_spec((1, _DH)),
              _full_spec((_DH, 10)), _full_spec((1, 10))],
    out_specs=_row_spec(_BN, 10),
    out_shape=jax.ShapeDtypeStruct((_N, 10), _f32),
)


def _build_edge_kernel(with_count):
    mesh = plsc.VectorSubcoreMesh(core_axis_name="c", subcore_axis_name="s")

    def body(*refs):
        if with_count:
            (t_hbm, h_hbm, src_hbm, dst_hbm, s_out, c_out,
             src_v, dst_v, h_v, rows_v, msg_v, ones_v, zer_v,
             s_sh, c_sh, sem_a, sem_b) = refs
        else:
            (t_hbm, h_hbm, src_hbm, dst_hbm, s_out,
             src_v, dst_v, h_v, rows_v, msg_v, zer_v,
             s_sh, sem_a, sem_b) = refs
        c = lax.axis_index("c")
        s = lax.axis_index("s")
        wid = s * 2 + c
        sems = (sem_a, sem_b)

        def zrow(i, _):
            zer_v[i, :] = jnp.zeros((_DH,), _f32)
            return 0
        lax.fori_loop(0, _RPS, zrow, 0)
        if with_count:
            def orow(i, _):
                ones_v[i, :] = jnp.full((_DH,), 1.0, _f32)
                return 0
            lax.fori_loop(0, _B, orow, 0)

        row0 = s * _RPS
        pltpu.sync_copy(zer_v, s_sh.at[pl.ds(row0, _RPS)])
        if with_count:
            pltpu.sync_copy(zer_v, c_sh.at[pl.ds(row0, _RPS)])
        plsc.subcore_barrier()

        rbase = wid * (_EPW // _B)

        def group(g, _):
            r0 = rbase + g * _CPG
            e0 = (rbase + g * _CPG) * _B
            pltpu.sync_copy(src_hbm.at[pl.ds(r0, _CPG)], src_v)
            pltpu.sync_copy(dst_hbm.at[pl.ds(r0, _CPG)], dst_v)
            pltpu.sync_copy(h_hbm.at[pl.ds(e0, _G)], h_v)
            pend = [None, None]
            pend[0] = pltpu.async_copy(t_hbm.at[src_v.at[0]],
                                       rows_v.at[0], sems[0])
            for j in range(_CPG):
                if j + 1 < _CPG:
                    pend[(j + 1) % 2] = pltpu.async_copy(
                        t_hbm.at[src_v.at[j + 1]],
                        rows_v.at[(j + 1) % 2], sems[(j + 1) % 2])
                pend[j % 2].wait()
                rows_j = rows_v.at[j % 2]

                @plsc.parallel_loop(0, _B, 1, unroll=2)
                def edge(e):
                    hrow = h_v[j * _B + e, :]
                    ab = rows_j[e, pl.ds(_KD * 2 * _DH // 2, 2 * _DH)]
                    acc, _zero = plsc.unpack(
                        ab, format=plsc.PackFormat.INTERLEAVED)
                    for p in range(_KD // 2):
                        abp = rows_j[e, pl.ds(2 * _DH * p, 2 * _DH)]
                        a, b = plsc.unpack(
                            abp, format=plsc.PackFormat.INTERLEAVED)
                        acc = (acc + hrow[2 * p] * a
                               + hrow[2 * p + 1] * b)
                    msg_v[e, :] = acc

                pltpu.sync_copy(msg_v, s_sh.at[dst_v.at[j]], add=True)
                if with_count:
                    pltpu.sync_copy(ones_v, c_sh.at[dst_v.at[j]], add=True)
            return 0
        lax.fori_loop(0, _NG, group, 0)

        plsc.subcore_barrier()
        pltpu.sync_copy(s_sh.at[pl.ds(row0, _RPS)],
                        s_out.at[c, pl.ds(row0, _RPS)])
        if with_count:
            pltpu.sync_copy(c_sh.at[pl.ds(row0, _RPS)],
                            c_out.at[c, pl.ds(row0, _RPS)])

    out_type = [jax.ShapeDtypeStruct((2, _NPAD, _DH), _f32)]
    scratch = [
        pltpu.VMEM((_CPG, _B), jnp.int32),
        pltpu.VMEM((_CPG, _B), jnp.int32),
        pltpu.VMEM((_G, _DH), _f32),
        pltpu.VMEM((2, _B, _TW), jnp.bfloat16),
        pltpu.VMEM((_B, _DH), _f32),
    ]
    if with_count:
        out_type.append(jax.ShapeDtypeStruct((2, _NPAD, _DH), _f32))
        scratch.append(pltpu.VMEM((_B, _DH), _f32))
    scratch.append(pltpu.VMEM((_RPS, _DH), _f32))
    scratch.append(pltpu.VMEM_SHARED((_NPAD, _DH), _f32))
    if with_count:
        scratch.append(pltpu.VMEM_SHARED((_NPAD, _DH), _f32))
    scratch.append(pltpu.SemaphoreType.DMA)
    scratch.append(pltpu.SemaphoreType.DMA)

    return pl.kernel(body, out_type=out_type, mesh=mesh,
                     scratch_types=scratch,
                     compiler_params=pltpu.CompilerParams(
                         use_tc_tiling_on_sc=False,
                         needs_layout_passes=False))


_edge_call_cnt = _build_edge_kernel(with_count=True)
_edge_call_nocnt = _build_edge_kernel(with_count=False)


def kernel(x, edge_index, edge_attr, en0_W1, en0_b1, en0_W2, en0_b2, root0,
           bias0, en1_W1, en1_b1, en1_W2, en1_b2, root1, bias1, cls_W, cls_b):
    src = edge_index[0].reshape(_E // _B, _B)
    dst = edge_index[1].reshape(_E // _B, _B)

    eye8 = jnp.eye(8, dtype=_f32)
    k0 = jnp.kron(eye8, en0_W1)
    k1 = jnp.kron(eye8, en1_W1)
    tb0 = jnp.tile(en0_b1, 8)[None, :]
    tb1 = jnp.tile(en1_b1, 8)[None, :]
    ea8 = edge_attr.reshape(_E8, 128)
    he0_r, he1_r = _edgenet_call(ea8, k0, tb0, k1, tb1)
    he0 = he0_r.reshape(_E, _DE)
    he1 = he1_r.reshape(_E, _DE)

    perm = jnp.array(_BF_PERM, dtype=jnp.int32)
    m0 = en0_W2.reshape(_DE, _DIN, _DH).transpose(1, 0, 2).reshape(_DIN, _KD * _DH)
    m0aug = jnp.concatenate([m0, en0_b2.reshape(_DIN, _DH),
                             jnp.zeros((_DIN, _DH), _f32)], axis=1)[:, perm]
    t0, r0 = _node_tab_call(x, m0aug, root0)

    s0, c0 = _edge_call_cnt(t0, he0, src, dst)

    m1 = en1_W2.reshape(_DE, _DH, _DH).transpose(1, 0, 2).reshape(_DH, _KD * _DH)
    m1aug = jnp.concatenate([m1, en1_b2.reshape(_DH, _DH),
                             jnp.zeros((_DH, _DH), _f32)], axis=1)[:, perm]
    t1, r1 = _mid_call(s0[0], s0[1], c0[0], c0[1], r0, bias0[None, :],
                       m1aug, root1)

    (s1,) = _edge_call_nocnt(t1, he1, src, dst)

    return _fin_call(s1[0], s1[1], c0[0], c0[1], r1, bias1[None, :],
                     cls_W, cls_b[None, :])


# speedup vs baseline: 5.3198x; 1.1398x over previous
import functools

import jax
import jax.numpy as jnp
from jax import lax
from jax.experimental import pallas as pl
from jax.experimental.pallas import tpu as pltpu
from jax.experimental.pallas import tpu_sc as plsc

_N = 10000
_E = 160000
_DIN = 128
_DH = 16
_DE = 16
_KD = 16
_TW = (_KD + 2) * _DH
_BF_PERM = tuple(
    (2 * p + half) * _DH + i
    for p in range(_TW // 32) for i in range(_DH) for half in (0, 1)
)

_E8 = _E // 8

_NW = 32
_EPW = _E // _NW
_B = 40
_CPG = 25
_G = 1000
_NG = _EPW // _G
_NSUB = 16
_NPAD = 10240
_RPS = _NPAD // _NSUB

_f32 = jnp.float32


def _elu(v):
    return jnp.where(v > 0, v, jnp.exp(jnp.minimum(v, 0.0)) - 1.0)


def _edgenet_body(ea_ref, k0_ref, b0_ref, k1_ref, b1_ref, h0_ref, h1_ref):
    ea = ea_ref[...]
    h0_ref[...] = _elu(jnp.dot(ea, k0_ref[...], preferred_element_type=_f32)
                       + b0_ref[...])
    h1_ref[...] = _elu(jnp.dot(ea, k1_ref[...], preferred_element_type=_f32)
                       + b1_ref[...])


def _node_tab_body(x_ref, m_ref, rt_ref, t_ref, r_ref):
    xb = x_ref[...]
    t_ref[...] = jnp.dot(xb, m_ref[...],
                         preferred_element_type=_f32).astype(jnp.bfloat16)
    r_ref[...] = jnp.dot(xb, rt_ref[...], preferred_element_type=_f32)


def _mid_body(sa_ref, sb_ref, ca_ref, cb_ref, r0_ref, b0_ref, m1_ref, rt1_ref,
              t1_ref, r1_ref):
    cnt = jnp.maximum(ca_ref[...] + cb_ref[...], 1.0)
    aggr = (sa_ref[...] + sb_ref[...]) / cnt
    h1 = _elu(aggr + r0_ref[...] + b0_ref[...])
    t1_ref[...] = jnp.dot(h1, m1_ref[...],
                          preferred_element_type=_f32).astype(jnp.bfloat16)
    r1_ref[...] = jnp.dot(h1, rt1_ref[...], preferred_element_type=_f32)


def _fin_body(sa_ref, sb_ref, ca_ref, cb_ref, r1_ref, b1_ref, w_ref, cb2_ref,
              o_ref):
    cnt = jnp.maximum(ca_ref[...] + cb_ref[...], 1.0)
    aggr = (sa_ref[...] + sb_ref[...]) / cnt
    h2 = _elu(aggr + r1_ref[...] + b1_ref[...])
    o_ref[...] = (jnp.dot(h2, w_ref[...], preferred_element_type=_f32)
                  + cb2_ref[...])


def _full_spec(shape):
    return pl.BlockSpec(shape, lambda i: (0,) * len(shape))


def _row_spec(bn, w):
    return pl.BlockSpec((bn, w), lambda i: (i, 0))


_BE = 2000
_BN = 2000

_edgenet_call = pl.pallas_call(
    _edgenet_body,
    grid=(_E8 // _BE,),
    in_specs=[_row_spec(_BE, 128), _full_spec((128, 128)), _full_spec((1, 128)),
              _full_spec((128, 128)), _full_spec((1, 128))],
    out_specs=[_row_spec(_BE, 128), _row_spec(_BE, 128)],
    out_shape=[jax.ShapeDtypeStruct((_E8, 128), _f32),
               jax.ShapeDtypeStruct((_E8, 128), _f32)],
)

_node_tab_call = pl.pallas_call(
    _node_tab_body,
    grid=(_N // _BN,),
    in_specs=[_row_spec(_BN, _DIN), _full_spec((_DIN, _TW)),
              _full_spec((_DIN, _DH))],
    out_specs=[_row_spec(_BN, _TW), _row_spec(_BN, _DH)],
    out_shape=[jax.ShapeDtypeStruct((_N, _TW), jnp.bfloat16),
               jax.ShapeDtypeStruct((_N, _DH), _f32)],
)

_mid_call = pl.pallas_call(
    _mid_body,
    grid=(_N // _BN,),
    in_specs=[_row_spec(_BN, _DH)] * 5 + [_full_spec((1, _DH)),
              _full_spec((_DH, _TW)), _full_spec((_DH, _DH))],
    out_specs=[_row_spec(_BN, _TW), _row_spec(_BN, _DH)],
    out_shape=[jax.ShapeDtypeStruct((_N, _TW), jnp.bfloat16),
               jax.ShapeDtypeStruct((_N, _DH), _f32)],
)

_fin_call = pl.pallas_call(
    _fin_body,
    grid=(_N // _BN,),
    in_specs=[_row_spec(_BN, _DH)] * 5 + [_full_spec((1, _DH)),
              _full_spec((_DH, 10)), _full_spec((1, 10))],
    out_specs=_row_spec(_BN, 10),
    out_shape=jax.ShapeDtypeStruct((_N, 10), _f32),
)


def _build_edge_kernel(with_count):
    mesh = plsc.VectorSubcoreMesh(core_axis_name="c", subcore_axis_name="s")

    def body(*refs):
        (t_hbm, h_hbm, src_hbm, dst_hbm, s_out,
         src_v, dst_v, h_v, rows_v, msg_v, zer_v,
         s_sh, sem_a, sem_b) = refs
        c = lax.axis_index("c")
        s = lax.axis_index("s")
        wid = s * 2 + c
        sems = (sem_a, sem_b)

        acc_w = 2 * _DH if with_count else _DH

        def zrow(i, _):
            zer_v[i, pl.ds(0, _DH)] = jnp.zeros((_DH,), _f32)
            if with_count:
                zer_v[i, pl.ds(_DH, _DH)] = jnp.zeros((_DH,), _f32)
            return 0
        lax.fori_loop(0, _RPS, zrow, 0)
        if with_count:
            def orow(i, _):
                msg_v[i, pl.ds(_DH, _DH)] = jnp.full((_DH,), 1.0, _f32)
                return 0
            lax.fori_loop(0, _B, orow, 0)

        row0 = s * _RPS
        pltpu.sync_copy(zer_v, s_sh.at[pl.ds(row0, _RPS)])
        plsc.subcore_barrier()

        rbase = wid * (_EPW // _B)

        def chunk(j, jv, rows_j):
            @plsc.parallel_loop(0, _B, 1, unroll=2)
            def edge(e):
                hrow = h_v[jv * _B + e, :]
                ab = rows_j[e, pl.ds(_KD * 2 * _DH // 2, 2 * _DH)]
                acc, _zero = plsc.unpack(
                    ab, format=plsc.PackFormat.INTERLEAVED)
                for p in range(_KD // 2):
                    abp = rows_j[e, pl.ds(2 * _DH * p, 2 * _DH)]
                    a, b = plsc.unpack(
                        abp, format=plsc.PackFormat.INTERLEAVED)
                    acc = (acc + hrow[2 * p] * a
                           + hrow[2 * p + 1] * b)
                msg_v[e, pl.ds(0, _DH)] = acc

            pltpu.sync_copy(msg_v, s_sh.at[dst_v.at[jv]], add=True)

        def group(g, _):
            r0 = rbase + g * _CPG
            e0 = (rbase + g * _CPG) * _B
            pltpu.sync_copy(src_hbm.at[pl.ds(r0, _CPG)], src_v)
            pltpu.sync_copy(dst_hbm.at[pl.ds(r0, _CPG)], dst_v)
            pltpu.sync_copy(h_hbm.at[pl.ds(e0, _G)], h_v)
            pltpu.async_copy(t_hbm.at[src_v.at[0]],
                             rows_v.at[0], sems[0]).wait()

            def pair(q, _q):
                jv0 = 2 * q
                h1 = pltpu.async_copy(t_hbm.at[src_v.at[jv0 + 1]],
                                      rows_v.at[1], sems[1])
                chunk(0, jv0, rows_v.at[0])
                h1.wait()
                h0 = pltpu.async_copy(t_hbm.at[src_v.at[jv0 + 2]],
                                      rows_v.at[0], sems[0])
                chunk(1, jv0 + 1, rows_v.at[1])
                h0.wait()
                return 0
            lax.fori_loop(0, (_CPG - 1) // 2, pair, 0)
            chunk(0, _CPG - 1, rows_v.at[0])
            return 0
        lax.fori_loop(0, _NG, group, 0)

        plsc.subcore_barrier()
        pltpu.sync_copy(s_sh.at[pl.ds(row0, _RPS)],
                        s_out.at[c, pl.ds(row0, _RPS)])

    acc_w = 2 * _DH if with_count else _DH
    out_type = [jax.ShapeDtypeStruct((2, _NPAD, acc_w), _f32)]
    scratch = [
        pltpu.VMEM((_CPG, _B), jnp.int32),
        pltpu.VMEM((_CPG, _B), jnp.int32),
        pltpu.VMEM((_G, _DH), _f32),
        pltpu.VMEM((2, _B, _TW), jnp.bfloat16),
        pltpu.VMEM((_B, acc_w), _f32),
        pltpu.VMEM((_RPS, acc_w), _f32),
        pltpu.VMEM_SHARED((_NPAD, acc_w), _f32),
        pltpu.SemaphoreType.DMA,
        pltpu.SemaphoreType.DMA,
    ]

    return pl.kernel(body, out_type=out_type, mesh=mesh,
                     scratch_types=scratch,
                     compiler_params=pltpu.CompilerParams(
                         use_tc_tiling_on_sc=False,
                         needs_layout_passes=False))


_edge_call_cnt = _build_edge_kernel(with_count=True)
_edge_call_nocnt = _build_edge_kernel(with_count=False)


def kernel(x, edge_index, edge_attr, en0_W1, en0_b1, en0_W2, en0_b2, root0,
           bias0, en1_W1, en1_b1, en1_W2, en1_b2, root1, bias1, cls_W, cls_b):
    src = edge_index[0].reshape(_E // _B, _B)
    dst = edge_index[1].reshape(_E // _B, _B)

    eye8 = jnp.eye(8, dtype=_f32)
    k0 = jnp.kron(eye8, en0_W1)
    k1 = jnp.kron(eye8, en1_W1)
    tb0 = jnp.tile(en0_b1, 8)[None, :]
    tb1 = jnp.tile(en1_b1, 8)[None, :]
    ea8 = edge_attr.reshape(_E8, 128)
    he0_r, he1_r = _edgenet_call(ea8, k0, tb0, k1, tb1)
    he0 = he0_r.reshape(_E, _DE)
    he1 = he1_r.reshape(_E, _DE)

    perm = jnp.array(_BF_PERM, dtype=jnp.int32)
    m0 = en0_W2.reshape(_DE, _DIN, _DH).transpose(1, 0, 2).reshape(_DIN, _KD * _DH)
    m0aug = jnp.concatenate([m0, en0_b2.reshape(_DIN, _DH),
                             jnp.zeros((_DIN, _DH), _f32)], axis=1)[:, perm]
    t0, r0 = _node_tab_call(x, m0aug, root0)

    (s0c,) = _edge_call_cnt(t0, he0, src, dst)
    s0 = s0c[:, :, :_DH]
    c0 = s0c[:, :, _DH:]

    m1 = en1_W2.reshape(_DE, _DH, _DH).transpose(1, 0, 2).reshape(_DH, _KD * _DH)
    m1aug = jnp.concatenate([m1, en1_b2.reshape(_DH, _DH),
                             jnp.zeros((_DH, _DH), _f32)], axis=1)[:, perm]
    t1, r1 = _mid_call(s0[0], s0[1], c0[0], c0[1], r0, bias0[None, :],
                       m1aug, root1)

    (s1,) = _edge_call_nocnt(t1, he1, src, dst)

    return _fin_call(s1[0], s1[1], c0[0], c0[1], r1, bias1[None, :],
                     cls_W, cls_b[None, :])


# speedup vs baseline: 5.5677x; 1.0466x over previous
import functools

import jax
import jax.numpy as jnp
from jax import lax
from jax.experimental import pallas as pl
from jax.experimental.pallas import tpu as pltpu
from jax.experimental.pallas import tpu_sc as plsc

_N = 10000
_E = 160000
_DIN = 128
_DH = 16
_DE = 16
_KD = 16
_TW = (_KD + 2) * _DH
_BF_PERM = tuple(
    (2 * p + half) * _DH + i
    for p in range(_TW // 32) for i in range(_DH) for half in (0, 1)
)

_E8 = _E // 8

_NW = 32
_EPW = _E // _NW
_B = 40
_CPG = 25
_G = 1000
_NG = _EPW // _G
_NSUB = 16
_NPAD = 10240
_RPS = _NPAD // _NSUB

_f32 = jnp.float32


def _elu(v):
    return jnp.where(v > 0, v, jnp.exp(jnp.minimum(v, 0.0)) - 1.0)


def _edgenet_body(ea_ref, k0_ref, b0_ref, k1_ref, b1_ref, h0_ref, h1_ref):
    ea = ea_ref[...]
    h0_ref[...] = _elu(jnp.dot(ea, k0_ref[...], preferred_element_type=_f32)
                       + b0_ref[...])
    h1_ref[...] = _elu(jnp.dot(ea, k1_ref[...], preferred_element_type=_f32)
                       + b1_ref[...])


def _node_tab_body(x_ref, m_ref, rt_ref, t_ref, r_ref):
    xb = x_ref[...]
    t_ref[...] = jnp.dot(xb, m_ref[...],
                         preferred_element_type=_f32).astype(jnp.bfloat16)
    r_ref[...] = jnp.dot(xb, rt_ref[...], preferred_element_type=_f32)


def _mid_body(sc_ref, r0_ref, b0_ref, m1_ref, rt1_ref,
              t1_ref, r1_ref):
    v = sc_ref[...]
    fused = v[0] + v[1]
    cnt = jnp.maximum(fused[:, _DH:], 1.0)
    aggr = fused[:, :_DH] / cnt
    h1 = _elu(aggr + r0_ref[...] + b0_ref[...])
    t1_ref[...] = jnp.dot(h1, m1_ref[...],
                          preferred_element_type=_f32).astype(jnp.bfloat16)
    r1_ref[...] = jnp.dot(h1, rt1_ref[...], preferred_element_type=_f32)


def _fin_body(s1_ref, sc_ref, r1_ref, b1_ref, w_ref, cb2_ref,
              o_ref):
    v = sc_ref[...]
    cnt = jnp.maximum(v[0, :, _DH:] + v[1, :, _DH:], 1.0)
    s1v = s1_ref[...]
    aggr = (s1v[0] + s1v[1]) / cnt
    h2 = _elu(aggr + r1_ref[...] + b1_ref[...])
    o_ref[...] = (jnp.dot(h2, w_ref[...], preferred_element_type=_f32)
                  + cb2_ref[...])


def _full_spec(shape):
    return pl.BlockSpec(shape, lambda i: (0,) * len(shape))


def _row_spec(bn, w):
    return pl.BlockSpec((bn, w), lambda i: (i, 0))


_BE = 2000
_BN = 2000

_edgenet_call = pl.pallas_call(
    _edgenet_body,
    grid=(_E8 // _BE,),
    in_specs=[_row_spec(_BE, 128), _full_spec((128, 128)), _full_spec((1, 128)),
              _full_spec((128, 128)), _full_spec((1, 128))],
    out_specs=[_row_spec(_BE, 128), _row_spec(_BE, 128)],
    out_shape=[jax.ShapeDtypeStruct((_E8, 128), _f32),
               jax.ShapeDtypeStruct((_E8, 128), _f32)],
)

_node_tab_call = pl.pallas_call(
    _node_tab_body,
    grid=(_N // _BN,),
    in_specs=[_row_spec(_BN, _DIN), _full_spec((_DIN, _TW)),
              _full_spec((_DIN, _DH))],
    out_specs=[_row_spec(_BN, _TW), _row_spec(_BN, _DH)],
    out_shape=[jax.ShapeDtypeStruct((_N, _TW), jnp.bfloat16),
               jax.ShapeDtypeStruct((_N, _DH), _f32)],
)

_mid_call = pl.pallas_call(
    _mid_body,
    grid=(_N // _BN,),
    in_specs=[pl.BlockSpec((2, _BN, 2 * _DH), lambda i: (0, i, 0)),
              _row_spec(_BN, _DH), _full_spec((1, _DH)),
              _full_spec((_DH, _TW)), _full_spec((_DH, _DH))],
    out_specs=[_row_spec(_BN, _TW), _row_spec(_BN, _DH)],
    out_shape=[jax.ShapeDtypeStruct((_N, _TW), jnp.bfloat16),
               jax.ShapeDtypeStruct((_N, _DH), _f32)],
)

_fin_call = pl.pallas_call(
    _fin_body,
    grid=(_N // _BN,),
    in_specs=[pl.BlockSpec((2, _BN, _DH), lambda i: (0, i, 0)),
              pl.BlockSpec((2, _BN, 2 * _DH), lambda i: (0, i, 0)),
              _row_spec(_BN, _DH), _full_spec((1, _DH)),
              _full_spec((_DH, 10)), _full_spec((1, 10))],
    out_specs=_row_spec(_BN, 10),
    out_shape=jax.ShapeDtypeStruct((_N, 10), _f32),
)


def _build_edge_kernel(with_count):
    mesh = plsc.VectorSubcoreMesh(core_axis_name="c", subcore_axis_name="s")

    def body(*refs):
        (t_hbm, h_hbm, src_hbm, dst_hbm, s_out,
         src_v, dst_v, h_v, rows_v, msg_v, zer_v,
         s_sh, sem_a, sem_b, sem_c, sem_d) = refs
        c = lax.axis_index("c")
        s = lax.axis_index("s")
        wid = s * 2 + c
        sems = (sem_a, sem_b)

        acc_w = 2 * _DH if with_count else _DH

        def zrow(i, _):
            zer_v[i, pl.ds(0, _DH)] = jnp.zeros((_DH,), _f32)
            if with_count:
                zer_v[i, pl.ds(_DH, _DH)] = jnp.zeros((_DH,), _f32)
            return 0
        lax.fori_loop(0, _RPS, zrow, 0)
        if with_count:
            def orow(i, _):
                msg_v[0, i, pl.ds(_DH, _DH)] = jnp.full((_DH,), 1.0, _f32)
                msg_v[1, i, pl.ds(_DH, _DH)] = jnp.full((_DH,), 1.0, _f32)
                return 0
            lax.fori_loop(0, _B, orow, 0)

        row0 = s * _RPS
        pltpu.sync_copy(zer_v, s_sh.at[pl.ds(row0, _RPS)])
        plsc.subcore_barrier()

        rbase = wid * (_EPW // _B)

        def chunk(jv, rows_j, msg_j, sc_sem):
            @plsc.parallel_loop(0, _B, 1, unroll=4)
            def edge(e):
                hrow = h_v[jv * _B + e, :]
                ab = rows_j[e, pl.ds(_KD * 2 * _DH // 2, 2 * _DH)]
                acc, _zero = plsc.unpack(
                    ab, format=plsc.PackFormat.INTERLEAVED)
                for p in range(_KD // 2):
                    abp = rows_j[e, pl.ds(2 * _DH * p, 2 * _DH)]
                    a, b = plsc.unpack(
                        abp, format=plsc.PackFormat.INTERLEAVED)
                    acc = (acc + hrow[2 * p] * a
                           + hrow[2 * p + 1] * b)
                msg_j[e, pl.ds(0, _DH)] = acc

            return pltpu.async_copy(msg_j, s_sh.at[dst_v.at[jv]],
                                    sc_sem, add=True)

        def group(g, _):
            r0 = rbase + g * _CPG
            e0 = (rbase + g * _CPG) * _B
            pltpu.sync_copy(src_hbm.at[pl.ds(r0, _CPG)], src_v)
            pltpu.sync_copy(dst_hbm.at[pl.ds(r0, _CPG)], dst_v)
            pltpu.sync_copy(h_hbm.at[pl.ds(e0, _G)], h_v)
            pltpu.async_copy(t_hbm.at[src_v.at[0]],
                             rows_v.at[0], sems[0]).wait()

            def pair(q, _q):
                jv0 = 2 * q
                h1 = pltpu.async_copy(t_hbm.at[src_v.at[jv0 + 1]],
                                      rows_v.at[1], sems[1])
                sca = chunk(jv0, rows_v.at[0], msg_v.at[0], sem_c)
                h1.wait()
                h0 = pltpu.async_copy(t_hbm.at[src_v.at[jv0 + 2]],
                                      rows_v.at[0], sems[0])
                scb = chunk(jv0 + 1, rows_v.at[1], msg_v.at[1], sem_d)
                sca.wait()
                scb.wait()
                h0.wait()
                return 0
            lax.fori_loop(0, (_CPG - 1) // 2, pair, 0)
            chunk(_CPG - 1, rows_v.at[0], msg_v.at[0], sem_c).wait()
            return 0
        lax.fori_loop(0, _NG, group, 0)

        plsc.subcore_barrier()
        pltpu.sync_copy(s_sh.at[pl.ds(row0, _RPS)],
                        s_out.at[c, pl.ds(row0, _RPS)])

    acc_w = 2 * _DH if with_count else _DH
    out_type = [jax.ShapeDtypeStruct((2, _NPAD, acc_w), _f32)]
    scratch = [
        pltpu.VMEM((_CPG, _B), jnp.int32),
        pltpu.VMEM((_CPG, _B), jnp.int32),
        pltpu.VMEM((_G, _DH), _f32),
        pltpu.VMEM((2, _B, _TW), jnp.bfloat16),
        pltpu.VMEM((2, _B, acc_w), _f32),
        pltpu.VMEM((_RPS, acc_w), _f32),
        pltpu.VMEM_SHARED((_NPAD, acc_w), _f32),
        pltpu.SemaphoreType.DMA,
        pltpu.SemaphoreType.DMA,
        pltpu.SemaphoreType.DMA,
        pltpu.SemaphoreType.DMA,
    ]

    return pl.kernel(body, out_type=out_type, mesh=mesh,
                     scratch_types=scratch,
                     compiler_params=pltpu.CompilerParams(
                         use_tc_tiling_on_sc=False,
                         needs_layout_passes=False))


_edge_call_cnt = _build_edge_kernel(with_count=True)
_edge_call_nocnt = _build_edge_kernel(with_count=False)


def kernel(x, edge_index, edge_attr, en0_W1, en0_b1, en0_W2, en0_b2, root0,
           bias0, en1_W1, en1_b1, en1_W2, en1_b2, root1, bias1, cls_W, cls_b):
    src = edge_index[0].reshape(_E // _B, _B)
    dst = edge_index[1].reshape(_E // _B, _B)

    eye8 = jnp.eye(8, dtype=_f32)
    k0 = jnp.kron(eye8, en0_W1)
    k1 = jnp.kron(eye8, en1_W1)
    tb0 = jnp.tile(en0_b1, 8)[None, :]
    tb1 = jnp.tile(en1_b1, 8)[None, :]
    ea8 = edge_attr.reshape(_E8, 128)
    he0_r, he1_r = _edgenet_call(ea8, k0, tb0, k1, tb1)
    he0 = he0_r.reshape(_E, _DE)
    he1 = he1_r.reshape(_E, _DE)

    perm = jnp.array(_BF_PERM, dtype=jnp.int32)
    m0 = en0_W2.reshape(_DE, _DIN, _DH).transpose(1, 0, 2).reshape(_DIN, _KD * _DH)
    m0aug = jnp.concatenate([m0, en0_b2.reshape(_DIN, _DH),
                             jnp.zeros((_DIN, _DH), _f32)], axis=1)[:, perm]
    t0, r0 = _node_tab_call(x, m0aug, root0)

    (s0c,) = _edge_call_cnt(t0, he0, src, dst)

    m1 = en1_W2.reshape(_DE, _DH, _DH).transpose(1, 0, 2).reshape(_DH, _KD * _DH)
    m1aug = jnp.concatenate([m1, en1_b2.reshape(_DH, _DH),
                             jnp.zeros((_DH, _DH), _f32)], axis=1)[:, perm]
    t1, r1 = _mid_call(s0c, r0, bias0[None, :], m1aug, root1)

    (s1,) = _edge_call_nocnt(t1, he1, src, dst)

    return _fin_call(s1, s0c, r1, bias1[None, :],
                     cls_W, cls_b[None, :])


# speedup vs baseline: 5.6418x; 1.0133x over previous
import functools

import jax
import jax.numpy as jnp
from jax import lax
from jax.experimental import pallas as pl
from jax.experimental.pallas import tpu as pltpu
from jax.experimental.pallas import tpu_sc as plsc

_N = 10000
_E = 160000
_DIN = 128
_DH = 16
_DE = 16
_KD = 16
_TW = (_KD + 2) * _DH
_BF_PERM = tuple(
    (2 * p + half) * _DH + i
    for p in range(_TW // 32) for i in range(_DH) for half in (0, 1)
)

_E8 = _E // 8

_NW = 32
_EPW = _E // _NW
_B = 40
_CPG = 25
_G = 1000
_NG = _EPW // _G
_NSUB = 16
_NPAD = 10240
_RPS = _NPAD // _NSUB

_f32 = jnp.float32


def _elu(v):
    return jnp.where(v > 0, v, jnp.exp(jnp.minimum(v, 0.0)) - 1.0)


def _prep_body(ea_ref, k0_ref, b0_ref, k1_ref, b1_ref, x_ref, m_ref, rt_ref,
               h0_ref, h1_ref, t_ref, r_ref):
    ea = ea_ref[...]
    h0_ref[...] = _elu(jnp.dot(ea, k0_ref[...], preferred_element_type=_f32)
                       + b0_ref[...])
    h1_ref[...] = _elu(jnp.dot(ea, k1_ref[...], preferred_element_type=_f32)
                       + b1_ref[...])
    xb = x_ref[...]
    t_ref[...] = jnp.dot(xb, m_ref[...],
                         preferred_element_type=_f32).astype(jnp.bfloat16)
    r_ref[...] = jnp.dot(xb, rt_ref[...], preferred_element_type=_f32)


def _mid_body(sc_ref, r0_ref, b0_ref, m1_ref, rt1_ref,
              t1_ref, r1_ref):
    v = sc_ref[...]
    fused = v[0] + v[1]
    cnt = jnp.maximum(fused[:, _DH:], 1.0)
    aggr = fused[:, :_DH] / cnt
    h1 = _elu(aggr + r0_ref[...] + b0_ref[...])
    t1_ref[...] = jnp.dot(h1, m1_ref[...],
                          preferred_element_type=_f32).astype(jnp.bfloat16)
    r1_ref[...] = jnp.dot(h1, rt1_ref[...], preferred_element_type=_f32)


def _fin_body(s1_ref, sc_ref, r1_ref, b1_ref, w_ref, cb2_ref,
              o_ref):
    v = sc_ref[...]
    cnt = jnp.maximum(v[0, :, _DH:] + v[1, :, _DH:], 1.0)
    s1v = s1_ref[...]
    aggr = (s1v[0] + s1v[1]) / cnt
    h2 = _elu(aggr + r1_ref[...] + b1_ref[...])
    o_ref[...] = (jnp.dot(h2, w_ref[...], preferred_element_type=_f32)
                  + cb2_ref[...])


def _full_spec(shape):
    return pl.BlockSpec(shape, lambda i: (0,) * len(shape))


def _row_spec(bn, w):
    return pl.BlockSpec((bn, w), lambda i: (i, 0))


_BE = 2000
_BX = 1000
_BN = 2000

_prep_call = pl.pallas_call(
    _prep_body,
    grid=(_E8 // _BE,),
    in_specs=[_row_spec(_BE, 128), _full_spec((128, 128)), _full_spec((1, 128)),
              _full_spec((128, 128)), _full_spec((1, 128)),
              _row_spec(_BX, _DIN), _full_spec((_DIN, _TW)),
              _full_spec((_DIN, _DH))],
    out_specs=[_row_spec(_BE, 128), _row_spec(_BE, 128),
               _row_spec(_BX, _TW), _row_spec(_BX, _DH)],
    out_shape=[jax.ShapeDtypeStruct((_E8, 128), _f32),
               jax.ShapeDtypeStruct((_E8, 128), _f32),
               jax.ShapeDtypeStruct((_N, _TW), jnp.bfloat16),
               jax.ShapeDtypeStruct((_N, _DH), _f32)],
)

_mid_call = pl.pallas_call(
    _mid_body,
    grid=(_N // _BN,),
    in_specs=[pl.BlockSpec((2, _BN, 2 * _DH), lambda i: (0, i, 0)),
              _row_spec(_BN, _DH), _full_spec((1, _DH)),
              _full_spec((_DH, _TW)), _full_spec((_DH, _DH))],
    out_specs=[_row_spec(_BN, _TW), _row_spec(_BN, _DH)],
    out_shape=[jax.ShapeDtypeStruct((_N, _TW), jnp.bfloat16),
               jax.ShapeDtypeStruct((_N, _DH), _f32)],
)

_fin_call = pl.pallas_call(
    _fin_body,
    grid=(_N // _BN,),
    in_specs=[pl.BlockSpec((2, _BN, _DH), lambda i: (0, i, 0)),
              pl.BlockSpec((2, _BN, 2 * _DH), lambda i: (0, i, 0)),
              _row_spec(_BN, _DH), _full_spec((1, _DH)),
              _full_spec((_DH, 10)), _full_spec((1, 10))],
    out_specs=_row_spec(_BN, 10),
    out_shape=jax.ShapeDtypeStruct((_N, 10), _f32),
)


def _build_edge_kernel(with_count):
    mesh = plsc.VectorSubcoreMesh(core_axis_name="c", subcore_axis_name="s")

    def body(*refs):
        (t_hbm, h_hbm, ei_hbm, s_out,
         src_v, dst_v, h_v, rows_v, msg_v, zer_v,
         s_sh, sem_a, sem_b, sem_c, sem_d) = refs
        c = lax.axis_index("c")
        s = lax.axis_index("s")
        wid = s * 2 + c
        sems = (sem_a, sem_b)

        acc_w = 2 * _DH if with_count else _DH

        def zrow(i, _):
            zer_v[i, pl.ds(0, _DH)] = jnp.zeros((_DH,), _f32)
            if with_count:
                zer_v[i, pl.ds(_DH, _DH)] = jnp.zeros((_DH,), _f32)
            return 0
        lax.fori_loop(0, _RPS, zrow, 0)
        if with_count:
            def orow(i, _):
                msg_v[0, i, pl.ds(_DH, _DH)] = jnp.full((_DH,), 1.0, _f32)
                msg_v[1, i, pl.ds(_DH, _DH)] = jnp.full((_DH,), 1.0, _f32)
                return 0
            lax.fori_loop(0, _B, orow, 0)

        row0 = s * _RPS
        pltpu.sync_copy(zer_v, s_sh.at[pl.ds(row0, _RPS)])
        plsc.subcore_barrier()

        rbase = wid * (_EPW // _B)

        def chunk(jv, rows_j, msg_j, sc_sem):
            @plsc.parallel_loop(0, _B, 1, unroll=4)
            def edge(e):
                hrow = h_v[jv * _B + e, :]
                ab = rows_j[e, pl.ds(_KD * 2 * _DH // 2, 2 * _DH)]
                acc, _zero = plsc.unpack(
                    ab, format=plsc.PackFormat.INTERLEAVED)
                for p in range(_KD // 2):
                    abp = rows_j[e, pl.ds(2 * _DH * p, 2 * _DH)]
                    a, b = plsc.unpack(
                        abp, format=plsc.PackFormat.INTERLEAVED)
                    acc = (acc + hrow[2 * p] * a
                           + hrow[2 * p + 1] * b)
                msg_j[e, pl.ds(0, _DH)] = acc

            return pltpu.async_copy(msg_j, s_sh.at[dst_v.at[jv]],
                                    sc_sem, add=True)

        def group(g, _):
            r0 = rbase + g * _CPG
            e0 = (rbase + g * _CPG) * _B
            pltpu.sync_copy(ei_hbm.at[0, pl.ds(r0, _CPG)], src_v)
            pltpu.sync_copy(ei_hbm.at[1, pl.ds(r0, _CPG)], dst_v)
            pltpu.sync_copy(h_hbm.at[pl.ds(e0, _G)], h_v)
            pltpu.async_copy(t_hbm.at[src_v.at[0]],
                             rows_v.at[0], sems[0]).wait()

            def pair(q, _q):
                jv0 = 2 * q
                h1 = pltpu.async_copy(t_hbm.at[src_v.at[jv0 + 1]],
                                      rows_v.at[1], sems[1])
                sca = chunk(jv0, rows_v.at[0], msg_v.at[0], sem_c)
                h1.wait()
                h0 = pltpu.async_copy(t_hbm.at[src_v.at[jv0 + 2]],
                                      rows_v.at[0], sems[0])
                scb = chunk(jv0 + 1, rows_v.at[1], msg_v.at[1], sem_d)
                sca.wait()
                scb.wait()
                h0.wait()
                return 0
            lax.fori_loop(0, (_CPG - 1) // 2, pair, 0)
            chunk(_CPG - 1, rows_v.at[0], msg_v.at[0], sem_c).wait()
            return 0
        lax.fori_loop(0, _NG, group, 0)

        plsc.subcore_barrier()
        pltpu.sync_copy(s_sh.at[pl.ds(row0, _RPS)],
                        s_out.at[c, pl.ds(row0, _RPS)])

    acc_w = 2 * _DH if with_count else _DH
    out_type = [jax.ShapeDtypeStruct((2, _NPAD, acc_w), _f32)]
    scratch = [
        pltpu.VMEM((_CPG, _B), jnp.int32),
        pltpu.VMEM((_CPG, _B), jnp.int32),
        pltpu.VMEM((_G, _DH), _f32),
        pltpu.VMEM((2, _B, _TW), jnp.bfloat16),
        pltpu.VMEM((2, _B, acc_w), _f32),
        pltpu.VMEM((_RPS, acc_w), _f32),
        pltpu.VMEM_SHARED((_NPAD, acc_w), _f32),
        pltpu.SemaphoreType.DMA,
        pltpu.SemaphoreType.DMA,
        pltpu.SemaphoreType.DMA,
        pltpu.SemaphoreType.DMA,
    ]

    return pl.kernel(body, out_type=out_type, mesh=mesh,
                     scratch_types=scratch,
                     compiler_params=pltpu.CompilerParams(
                         use_tc_tiling_on_sc=False,
                         needs_layout_passes=False))


_edge_call_cnt = _build_edge_kernel(with_count=True)
_edge_call_nocnt = _build_edge_kernel(with_count=False)


def kernel(x, edge_index, edge_attr, en0_W1, en0_b1, en0_W2, en0_b2, root0,
           bias0, en1_W1, en1_b1, en1_W2, en1_b2, root1, bias1, cls_W, cls_b):
    ei3 = edge_index.reshape(2, _E // _B, _B)

    eye8 = jnp.eye(8, dtype=_f32)
    k0 = jnp.kron(eye8, en0_W1)
    k1 = jnp.kron(eye8, en1_W1)
    tb0 = jnp.tile(en0_b1, 8)[None, :]
    tb1 = jnp.tile(en1_b1, 8)[None, :]
    ea8 = edge_attr.reshape(_E8, 128)
    perm = jnp.array(_BF_PERM, dtype=jnp.int32)
    m0 = en0_W2.reshape(_DE, _DIN, _DH).transpose(1, 0, 2).reshape(_DIN, _KD * _DH)
    m0aug = jnp.concatenate([m0, en0_b2.reshape(_DIN, _DH),
                             jnp.zeros((_DIN, _DH), _f32)], axis=1)[:, perm]
    he0_r, he1_r, t0, r0 = _prep_call(ea8, k0, tb0, k1, tb1, x, m0aug, root0)
    he0 = he0_r.reshape(_E, _DE)
    he1 = he1_r.reshape(_E, _DE)

    (s0c,) = _edge_call_cnt(t0, he0, ei3)

    m1 = en1_W2.reshape(_DE, _DH, _DH).transpose(1, 0, 2).reshape(_DH, _KD * _DH)
    m1aug = jnp.concatenate([m1, en1_b2.reshape(_DH, _DH),
                             jnp.zeros((_DH, _DH), _f32)], axis=1)[:, perm]
    t1, r1 = _mid_call(s0c, r0, bias0[None, :], m1aug, root1)

    (s1,) = _edge_call_nocnt(t1, he1, ei3)

    return _fin_call(s1, s0c, r1, bias1[None, :],
                     cls_W, cls_b[None, :])


# speedup vs baseline: 5.8536x; 1.0375x over previous
import functools

import jax
import jax.numpy as jnp
from jax import lax
from jax.experimental import pallas as pl
from jax.experimental.pallas import tpu as pltpu
from jax.experimental.pallas import tpu_sc as plsc

_N = 10000
_E = 160000
_DIN = 128
_DH = 16
_DE = 16
_KD = 16
_TW = (_KD + 2) * _DH
_BF_PERM = tuple(
    (2 * p + half) * _DH + i
    for p in range(_TW // 32) for i in range(_DH) for half in (0, 1)
)

_E8 = _E // 8

_NW = 32
_EPW = _E // _NW
_B = 40
_CPG = 25
_G = 1000
_NG = _EPW // _G
_NSUB = 16
_NPAD = 10240
_RPS = _NPAD // _NSUB

_f32 = jnp.float32


def _elu(v):
    return jnp.where(v > 0, v, jnp.exp(jnp.minimum(v, 0.0)) - 1.0)


def _prep_body(ea_ref, k0_ref, b0_ref, k1_ref, b1_ref, x_ref, m_ref, rt_ref,
               h0_ref, h1_ref, t_ref, r_ref):
    ea = ea_ref[...]
    h0_ref[...] = _elu(jnp.dot(ea, k0_ref[...], preferred_element_type=_f32)
                       + b0_ref[...])
    h1_ref[...] = _elu(jnp.dot(ea, k1_ref[...], preferred_element_type=_f32)
                       + b1_ref[...])
    xb = x_ref[...]
    t_ref[...] = jnp.dot(xb, m_ref[...],
                         preferred_element_type=_f32).astype(jnp.bfloat16)
    r_ref[...] = jnp.dot(xb, rt_ref[...], preferred_element_type=_f32)


def _mid_body(sc_ref, r0_ref, b0_ref, m1_ref, rt1_ref,
              t1_ref, r1_ref):
    v = sc_ref[...]
    fused = v[0] + v[1]
    cnt = jnp.maximum(fused[:, _DH:], 1.0)
    aggr = fused[:, :_DH] / cnt
    h1 = _elu(aggr + r0_ref[...] + b0_ref[...])
    t1_ref[...] = jnp.dot(h1, m1_ref[...],
                          preferred_element_type=_f32).astype(jnp.bfloat16)
    r1_ref[...] = jnp.dot(h1, rt1_ref[...], preferred_element_type=_f32)


def _fin_body(s1_ref, sc_ref, r1_ref, b1_ref, w_ref, cb2_ref,
              o_ref):
    v = sc_ref[...]
    cnt = jnp.maximum(v[0, :, _DH:] + v[1, :, _DH:], 1.0)
    s1v = s1_ref[...]
    aggr = (s1v[0] + s1v[1]) / cnt
    h2 = _elu(aggr + r1_ref[...] + b1_ref[...])
    o_ref[...] = (jnp.dot(h2, w_ref[...], preferred_element_type=_f32)
                  + cb2_ref[...])


def _full_spec(shape):
    return pl.BlockSpec(shape, lambda i: (0,) * len(shape))


def _row_spec(bn, w):
    return pl.BlockSpec((bn, w), lambda i: (i, 0))


_BE = 2000
_BX = 1000
_BN = 2000

_prep_call = pl.pallas_call(
    _prep_body,
    grid=(_E8 // _BE,),
    in_specs=[_row_spec(_BE, 128), _full_spec((128, 128)), _full_spec((1, 128)),
              _full_spec((128, 128)), _full_spec((1, 128)),
              _row_spec(_BX, _DIN), _full_spec((_DIN, _TW)),
              _full_spec((_DIN, _DH))],
    out_specs=[_row_spec(_BE, 128), _row_spec(_BE, 128),
               _row_spec(_BX, _TW), _row_spec(_BX, _DH)],
    out_shape=[jax.ShapeDtypeStruct((_E8, 128), _f32),
               jax.ShapeDtypeStruct((_E8, 128), _f32),
               jax.ShapeDtypeStruct((_N, _TW), jnp.bfloat16),
               jax.ShapeDtypeStruct((_N, _DH), _f32)],
)

_mid_call = pl.pallas_call(
    _mid_body,
    grid=(_N // _BN,),
    in_specs=[pl.BlockSpec((2, _BN, 2 * _DH), lambda i: (0, i, 0)),
              _row_spec(_BN, _DH), _full_spec((1, _DH)),
              _full_spec((_DH, _TW)), _full_spec((_DH, _DH))],
    out_specs=[_row_spec(_BN, _TW), _row_spec(_BN, _DH)],
    out_shape=[jax.ShapeDtypeStruct((_N, _TW), jnp.bfloat16),
               jax.ShapeDtypeStruct((_N, _DH), _f32)],
)

_fin_call = pl.pallas_call(
    _fin_body,
    grid=(_N // _BN,),
    in_specs=[pl.BlockSpec((2, _BN, _DH), lambda i: (0, i, 0)),
              pl.BlockSpec((2, _BN, 2 * _DH), lambda i: (0, i, 0)),
              _row_spec(_BN, _DH), _full_spec((1, _DH)),
              _full_spec((_DH, 10)), _full_spec((1, 10))],
    out_specs=_row_spec(_BN, 10),
    out_shape=jax.ShapeDtypeStruct((_N, 10), _f32),
)


def _build_edge_kernel(with_count):
    mesh = plsc.VectorSubcoreMesh(core_axis_name="c", subcore_axis_name="s")

    def body(*refs):
        (t_hbm, h_hbm, ei_hbm, s_out,
         src_v, dst_v, h_v, rows_v, msg_v, zer_v,
         s_sh, sem_a, sem_b, sem_c, sem_d, sem_i) = refs
        c = lax.axis_index("c")
        s = lax.axis_index("s")
        wid = s * 2 + c
        sems = (sem_a, sem_b)

        acc_w = 2 * _DH if with_count else _DH

        def zrow(i, _):
            zer_v[i, pl.ds(0, _DH)] = jnp.zeros((_DH,), _f32)
            if with_count:
                zer_v[i, pl.ds(_DH, _DH)] = jnp.zeros((_DH,), _f32)
            return 0
        lax.fori_loop(0, _RPS, zrow, 0)
        if with_count:
            def orow(i, _):
                msg_v[0, i, pl.ds(_DH, _DH)] = jnp.full((_DH,), 1.0, _f32)
                msg_v[1, i, pl.ds(_DH, _DH)] = jnp.full((_DH,), 1.0, _f32)
                return 0
            lax.fori_loop(0, _B, orow, 0)

        row0 = s * _RPS
        pltpu.sync_copy(zer_v, s_sh.at[pl.ds(row0, _RPS)])
        plsc.subcore_barrier()

        rbase = wid * (_EPW // _B)

        def chunk(gb, jv, rows_j, msg_j, sc_sem):
            @plsc.parallel_loop(0, _B, 1, unroll=4)
            def edge(e):
                hrow = h_v[gb, jv * _B + e, :]
                ab = rows_j[e, pl.ds(_KD * 2 * _DH // 2, 2 * _DH)]
                acc, _zero = plsc.unpack(
                    ab, format=plsc.PackFormat.INTERLEAVED)
                for p in range(_KD // 2):
                    abp = rows_j[e, pl.ds(2 * _DH * p, 2 * _DH)]
                    a, b = plsc.unpack(
                        abp, format=plsc.PackFormat.INTERLEAVED)
                    acc = (acc + hrow[2 * p] * a
                           + hrow[2 * p + 1] * b)
                msg_j[e, pl.ds(0, _DH)] = acc

            return pltpu.async_copy(msg_j, s_sh.at[dst_v.at[gb, jv]],
                                    sc_sem, add=True)

        def icopies(gi, buf, start):
            r0i = rbase + gi * _CPG
            e0i = r0i * _B
            op = pltpu.async_copy if start else (
                lambda s, d, m: pltpu.make_async_copy(s, d, m).wait())
            op(ei_hbm.at[0, pl.ds(r0i, _CPG)], src_v.at[buf], sem_i)
            op(ei_hbm.at[1, pl.ds(r0i, _CPG)], dst_v.at[buf], sem_i)
            op(h_hbm.at[pl.ds(e0i, _G)], h_v.at[buf], sem_i)

        icopies(0, 0, True)

        def group(g, _):
            gb = g % 2
            icopies(g, gb, False)

            @pl.when(g + 1 < _NG)
            def _prefetch():
                icopies(g + 1, (g + 1) % 2, True)

            srcs = src_v.at[gb]
            pltpu.async_copy(t_hbm.at[srcs.at[0]],
                             rows_v.at[0], sems[0]).wait()

            def pair(q, _q):
                jv0 = 2 * q
                h1 = pltpu.async_copy(t_hbm.at[srcs.at[jv0 + 1]],
                                      rows_v.at[1], sems[1])
                sca = chunk(gb, jv0, rows_v.at[0], msg_v.at[0], sem_c)
                h1.wait()
                h0 = pltpu.async_copy(t_hbm.at[srcs.at[jv0 + 2]],
                                      rows_v.at[0], sems[0])
                scb = chunk(gb, jv0 + 1, rows_v.at[1], msg_v.at[1], sem_d)
                sca.wait()
                h0.wait()
                scb.wait()
                return 0
            lax.fori_loop(0, (_CPG - 1) // 2, pair, 0)
            chunk(gb, _CPG - 1, rows_v.at[0], msg_v.at[0], sem_c).wait()
            return 0
        lax.fori_loop(0, _NG, group, 0)

        plsc.subcore_barrier()
        pltpu.sync_copy(s_sh.at[pl.ds(row0, _RPS)],
                        s_out.at[c, pl.ds(row0, _RPS)])

    acc_w = 2 * _DH if with_count else _DH
    out_type = [jax.ShapeDtypeStruct((2, _NPAD, acc_w), _f32)]
    scratch = [
        pltpu.VMEM((2, _CPG, _B), jnp.int32),
        pltpu.VMEM((2, _CPG, _B), jnp.int32),
        pltpu.VMEM((2, _G, _DH), _f32),
        pltpu.VMEM((2, _B, _TW), jnp.bfloat16),
        pltpu.VMEM((2, _B, acc_w), _f32),
        pltpu.VMEM((_RPS, acc_w), _f32),
        pltpu.VMEM_SHARED((_NPAD, acc_w), _f32),
        pltpu.SemaphoreType.DMA,
        pltpu.SemaphoreType.DMA,
        pltpu.SemaphoreType.DMA,
        pltpu.SemaphoreType.DMA,
        pltpu.SemaphoreType.DMA,
    ]

    return pl.kernel(body, out_type=out_type, mesh=mesh,
                     scratch_types=scratch,
                     compiler_params=pltpu.CompilerParams(
                         use_tc_tiling_on_sc=False,
                         needs_layout_passes=False))


_edge_call_cnt = _build_edge_kernel(with_count=True)
_edge_call_nocnt = _build_edge_kernel(with_count=False)


def kernel(x, edge_index, edge_attr, en0_W1, en0_b1, en0_W2, en0_b2, root0,
           bias0, en1_W1, en1_b1, en1_W2, en1_b2, root1, bias1, cls_W, cls_b):
    ei3 = edge_index.reshape(2, _E // _B, _B)

    eye8 = jnp.eye(8, dtype=_f32)
    k0 = jnp.kron(eye8, en0_W1)
    k1 = jnp.kron(eye8, en1_W1)
    tb0 = jnp.tile(en0_b1, 8)[None, :]
    tb1 = jnp.tile(en1_b1, 8)[None, :]
    ea8 = edge_attr.reshape(_E8, 128)
    perm = jnp.array(_BF_PERM, dtype=jnp.int32)
    m0 = en0_W2.reshape(_DE, _DIN, _DH).transpose(1, 0, 2).reshape(_DIN, _KD * _DH)
    m0aug = jnp.concatenate([m0, en0_b2.reshape(_DIN, _DH),
                             jnp.zeros((_DIN, _DH), _f32)], axis=1)[:, perm]
    he0_r, he1_r, t0, r0 = _prep_call(ea8, k0, tb0, k1, tb1, x, m0aug, root0)
    he0 = he0_r.reshape(_E, _DE)
    he1 = he1_r.reshape(_E, _DE)

    (s0c,) = _edge_call_cnt(t0, he0, ei3)

    m1 = en1_W2.reshape(_DE, _DH, _DH).transpose(1, 0, 2).reshape(_DH, _KD * _DH)
    m1aug = jnp.concatenate([m1, en1_b2.reshape(_DH, _DH),
                             jnp.zeros((_DH, _DH), _f32)], axis=1)[:, perm]
    t1, r1 = _mid_call(s0c, r0, bias0[None, :], m1aug, root1)

    (s1,) = _edge_call_nocnt(t1, he1, ei3)

    return _fin_call(s1, s0c, r1, bias1[None, :],
                     cls_W, cls_b[None, :])


# speedup vs baseline: 5.8551x; 1.0003x over previous
import functools

import jax
import jax.numpy as jnp
from jax import lax
from jax.experimental import pallas as pl
from jax.experimental.pallas import tpu as pltpu
from jax.experimental.pallas import tpu_sc as plsc

_N = 10000
_E = 160000
_DIN = 128
_DH = 16
_DE = 16
_KD = 16
_TW = (_KD + 2) * _DH
_BF_PERM = tuple(
    (2 * p + half) * _DH + i
    for p in range(_TW // 32) for i in range(_DH) for half in (0, 1)
)

_E8 = _E // 8

_NW = 32
_EPW = _E // _NW
_B = 40
_CPG = 25
_G = 1000
_NG = _EPW // _G
_NSUB = 16
_NPAD = 10240
_RPS = _NPAD // _NSUB

_f32 = jnp.float32


def _elu(v):
    return jnp.where(v > 0, v, jnp.exp(jnp.minimum(v, 0.0)) - 1.0)


def _prep_body(ea_ref, k0_ref, b0_ref, k1_ref, b1_ref, x_ref, m_ref, rt_ref,
               h0_ref, h1_ref, t_ref, r_ref):
    ea = ea_ref[...]
    h0_ref[...] = _elu(jnp.dot(ea, k0_ref[...], preferred_element_type=_f32)
                       + b0_ref[...])
    h1_ref[...] = _elu(jnp.dot(ea, k1_ref[...], preferred_element_type=_f32)
                       + b1_ref[...])
    xb = x_ref[...]
    t_ref[...] = jnp.dot(xb, m_ref[...],
                         preferred_element_type=_f32).astype(jnp.bfloat16)
    r_ref[...] = jnp.dot(xb, rt_ref[...], preferred_element_type=_f32)


def _mid_body(sc_ref, r0_ref, b0_ref, m1_ref, rt1_ref,
              t1_ref, r1_ref):
    v = sc_ref[...]
    fused = v[0] + v[1]
    cnt = jnp.maximum(fused[:, _DH:], 1.0)
    aggr = fused[:, :_DH] / cnt
    h1 = _elu(aggr + r0_ref[...] + b0_ref[...])
    t1_ref[...] = jnp.dot(h1, m1_ref[...],
                          preferred_element_type=_f32).astype(jnp.bfloat16)
    r1_ref[...] = jnp.dot(h1, rt1_ref[...], preferred_element_type=_f32)


def _fin_body(s1_ref, sc_ref, r1_ref, b1_ref, w_ref, cb2_ref,
              o_ref):
    v = sc_ref[...]
    cnt = jnp.maximum(v[0, :, _DH:] + v[1, :, _DH:], 1.0)
    s1v = s1_ref[...]
    aggr = (s1v[0] + s1v[1]) / cnt
    h2 = _elu(aggr + r1_ref[...] + b1_ref[...])
    o_ref[...] = (jnp.dot(h2, w_ref[...], preferred_element_type=_f32)
                  + cb2_ref[...])


def _full_spec(shape):
    return pl.BlockSpec(shape, lambda i: (0,) * len(shape))


def _row_spec(bn, w):
    return pl.BlockSpec((bn, w), lambda i: (i, 0))


_BE = 2000
_BX = 1000
_BN = 2000

_prep_call = pl.pallas_call(
    _prep_body,
    grid=(_E8 // _BE,),
    in_specs=[_row_spec(_BE, 128), _full_spec((128, 128)), _full_spec((1, 128)),
              _full_spec((128, 128)), _full_spec((1, 128)),
              _row_spec(_BX, _DIN), _full_spec((_DIN, _TW)),
              _full_spec((_DIN, _DH))],
    out_specs=[_row_spec(_BE, 128), _row_spec(_BE, 128),
               _row_spec(_BX, _TW), _row_spec(_BX, _DH)],
    out_shape=[jax.ShapeDtypeStruct((_E8, 128), _f32),
               jax.ShapeDtypeStruct((_E8, 128), _f32),
               jax.ShapeDtypeStruct((_N, _TW), jnp.bfloat16),
               jax.ShapeDtypeStruct((_N, _DH), _f32)],
)

_mid_call = pl.pallas_call(
    _mid_body,
    grid=(_N // _BN,),
    in_specs=[pl.BlockSpec((2, _BN, 2 * _DH), lambda i: (0, i, 0)),
              _row_spec(_BN, _DH), _full_spec((1, _DH)),
              _full_spec((_DH, _TW)), _full_spec((_DH, _DH))],
    out_specs=[_row_spec(_BN, _TW), _row_spec(_BN, _DH)],
    out_shape=[jax.ShapeDtypeStruct((_N, _TW), jnp.bfloat16),
               jax.ShapeDtypeStruct((_N, _DH), _f32)],
)

_fin_call = pl.pallas_call(
    _fin_body,
    grid=(_N // _BN,),
    in_specs=[pl.BlockSpec((2, _BN, _DH), lambda i: (0, i, 0)),
              pl.BlockSpec((2, _BN, 2 * _DH), lambda i: (0, i, 0)),
              _row_spec(_BN, _DH), _full_spec((1, _DH)),
              _full_spec((_DH, 10)), _full_spec((1, 10))],
    out_specs=_row_spec(_BN, 10),
    out_shape=jax.ShapeDtypeStruct((_N, 10), _f32),
)


def _build_edge_kernel(with_count):
    mesh = plsc.VectorSubcoreMesh(core_axis_name="c", subcore_axis_name="s")

    def body(*refs):
        (t_hbm, h_hbm, ei_hbm, s_out,
         src_v, dst_v, h_v, rows_v, msg_v, zer_v,
         s_sh, sem_a, sem_b, sem_c, sem_d, sem_i) = refs
        c = lax.axis_index("c")
        s = lax.axis_index("s")
        wid = s * 2 + c
        sems = (sem_a, sem_b)

        acc_w = 2 * _DH if with_count else _DH

        def zrow(i, _):
            zer_v[i, pl.ds(0, _DH)] = jnp.zeros((_DH,), _f32)
            if with_count:
                zer_v[i, pl.ds(_DH, _DH)] = jnp.zeros((_DH,), _f32)
            return 0
        lax.fori_loop(0, _RPS, zrow, 0)
        if with_count:
            def orow(i, _):
                msg_v[0, i, pl.ds(_DH, _DH)] = jnp.full((_DH,), 1.0, _f32)
                msg_v[1, i, pl.ds(_DH, _DH)] = jnp.full((_DH,), 1.0, _f32)
                return 0
            lax.fori_loop(0, _B, orow, 0)

        row0 = s * _RPS
        pltpu.sync_copy(zer_v, s_sh.at[pl.ds(row0, _RPS)])
        plsc.subcore_barrier()

        rbase = wid * (_EPW // _B)

        def chunk(gb, jv, rows_j, msg_j, sc_sem):
            @plsc.parallel_loop(0, _B, 1, unroll=8)
            def edge(e):
                hrow = h_v[gb, jv * _B + e, :]
                ab = rows_j[e, pl.ds(_KD * 2 * _DH // 2, 2 * _DH)]
                acc, _zero = plsc.unpack(
                    ab, format=plsc.PackFormat.INTERLEAVED)
                for p in range(_KD // 2):
                    abp = rows_j[e, pl.ds(2 * _DH * p, 2 * _DH)]
                    a, b = plsc.unpack(
                        abp, format=plsc.PackFormat.INTERLEAVED)
                    acc = (acc + hrow[2 * p] * a
                           + hrow[2 * p + 1] * b)
                msg_j[e, pl.ds(0, _DH)] = acc

            return pltpu.async_copy(msg_j, s_sh.at[dst_v.at[gb, jv]],
                                    sc_sem, add=True)

        def icopies(gi, buf, start):
            r0i = rbase + gi * _CPG
            e0i = r0i * _B
            op = pltpu.async_copy if start else (
                lambda s, d, m: pltpu.make_async_copy(s, d, m).wait())
            op(ei_hbm.at[0, pl.ds(r0i, _CPG)], src_v.at[buf], sem_i)
            op(ei_hbm.at[1, pl.ds(r0i, _CPG)], dst_v.at[buf], sem_i)
            op(h_hbm.at[pl.ds(e0i, _G)], h_v.at[buf], sem_i)

        icopies(0, 0, True)

        def group(g, _):
            gb = g % 2
            icopies(g, gb, False)

            @pl.when(g + 1 < _NG)
            def _prefetch():
                icopies(g + 1, (g + 1) % 2, True)

            srcs = src_v.at[gb]
            pltpu.async_copy(t_hbm.at[srcs.at[0]],
                             rows_v.at[0], sems[0]).wait()

            def pair(q, _q):
                jv0 = 2 * q
                h1 = pltpu.async_copy(t_hbm.at[srcs.at[jv0 + 1]],
                                      rows_v.at[1], sems[1])
                sca = chunk(gb, jv0, rows_v.at[0], msg_v.at[0], sem_c)
                h1.wait()
                h0 = pltpu.async_copy(t_hbm.at[srcs.at[jv0 + 2]],
                                      rows_v.at[0], sems[0])
                scb = chunk(gb, jv0 + 1, rows_v.at[1], msg_v.at[1], sem_d)
                sca.wait()
                h0.wait()
                scb.wait()
                return 0
            lax.fori_loop(0, (_CPG - 1) // 2, pair, 0)
            chunk(gb, _CPG - 1, rows_v.at[0], msg_v.at[0], sem_c).wait()
            return 0
        lax.fori_loop(0, _NG, group, 0)

        plsc.subcore_barrier()
        pltpu.sync_copy(s_sh.at[pl.ds(row0, _RPS)],
                        s_out.at[c, pl.ds(row0, _RPS)])

    acc_w = 2 * _DH if with_count else _DH
    out_type = [jax.ShapeDtypeStruct((2, _NPAD, acc_w), _f32)]
    scratch = [
        pltpu.VMEM((2, _CPG, _B), jnp.int32),
        pltpu.VMEM((2, _CPG, _B), jnp.int32),
        pltpu.VMEM((2, _G, _DH), _f32),
        pltpu.VMEM((2, _B, _TW), jnp.bfloat16),
        pltpu.VMEM((2, _B, acc_w), _f32),
        pltpu.VMEM((_RPS, acc_w), _f32),
        pltpu.VMEM_SHARED((_NPAD, acc_w), _f32),
        pltpu.SemaphoreType.DMA,
        pltpu.SemaphoreType.DMA,
        pltpu.SemaphoreType.DMA,
        pltpu.SemaphoreType.DMA,
        pltpu.SemaphoreType.DMA,
    ]

    return pl.kernel(body, out_type=out_type, mesh=mesh,
                     scratch_types=scratch,
                     compiler_params=pltpu.CompilerParams(
                         use_tc_tiling_on_sc=False,
                         needs_layout_passes=False))


_edge_call_cnt = _build_edge_kernel(with_count=True)
_edge_call_nocnt = _build_edge_kernel(with_count=False)


def kernel(x, edge_index, edge_attr, en0_W1, en0_b1, en0_W2, en0_b2, root0,
           bias0, en1_W1, en1_b1, en1_W2, en1_b2, root1, bias1, cls_W, cls_b):
    ei3 = edge_index.reshape(2, _E // _B, _B)

    eye8 = jnp.eye(8, dtype=_f32)
    k0 = jnp.kron(eye8, en0_W1)
    k1 = jnp.kron(eye8, en1_W1)
    tb0 = jnp.tile(en0_b1, 8)[None, :]
    tb1 = jnp.tile(en1_b1, 8)[None, :]
    ea8 = edge_attr.reshape(_E8, 128)
    perm = jnp.array(_BF_PERM, dtype=jnp.int32)
    m0 = en0_W2.reshape(_DE, _DIN, _DH).transpose(1, 0, 2).reshape(_DIN, _KD * _DH)
    m0aug = jnp.concatenate([m0, en0_b2.reshape(_DIN, _DH),
                             jnp.zeros((_DIN, _DH), _f32)], axis=1)[:, perm]
    he0_r, he1_r, t0, r0 = _prep_call(ea8, k0, tb0, k1, tb1, x, m0aug, root0)
    he0 = he0_r.reshape(_E, _DE)
    he1 = he1_r.reshape(_E, _DE)

    (s0c,) = _edge_call_cnt(t0, he0, ei3)

    m1 = en1_W2.reshape(_DE, _DH, _DH).transpose(1, 0, 2).reshape(_DH, _KD * _DH)
    m1aug = jnp.concatenate([m1, en1_b2.reshape(_DH, _DH),
                             jnp.zeros((_DH, _DH), _f32)], axis=1)[:, perm]
    t1, r1 = _mid_call(s0c, r0, bias0[None, :], m1aug, root1)

    (s1,) = _edge_call_nocnt(t1, he1, ei3)

    return _fin_call(s1, s0c, r1, bias1[None, :],
                     cls_W, cls_b[None, :])


# speedup vs baseline: 5.8593x; 1.0007x over previous
import functools

import jax
import jax.numpy as jnp
from jax import lax
from jax.experimental import pallas as pl
from jax.experimental.pallas import tpu as pltpu
from jax.experimental.pallas import tpu_sc as plsc

_N = 10000
_E = 160000
_DIN = 128
_DH = 16
_DE = 16
_KD = 16
_TW = (_KD + 2) * _DH
_BF_PERM = tuple(
    (2 * p + half) * _DH + i
    for p in range(_TW // 32) for i in range(_DH) for half in (0, 1)
)

_E8 = _E // 8

_NW = 32
_EPW = _E // _NW
_B = 40
_CPG = 25
_G = 1000
_NG = _EPW // _G
_NSUB = 16
_NPAD = 10240
_RPS = _NPAD // _NSUB

_f32 = jnp.float32


def _elu(v):
    return jnp.where(v > 0, v, jnp.exp(jnp.minimum(v, 0.0)) - 1.0)


def _prep_body(ea_ref, k0_ref, b0_ref, k1_ref, b1_ref, x_ref, m_ref, rt_ref,
               h0_ref, h1_ref, t_ref, r_ref):
    ea = ea_ref[...]
    h0_ref[...] = _elu(jnp.dot(ea, k0_ref[...], preferred_element_type=_f32)
                       + b0_ref[...])
    h1_ref[...] = _elu(jnp.dot(ea, k1_ref[...], preferred_element_type=_f32)
                       + b1_ref[...])
    xb = x_ref[...]
    t_ref[...] = jnp.dot(xb, m_ref[...],
                         preferred_element_type=_f32).astype(jnp.bfloat16)
    r_ref[...] = jnp.dot(xb, rt_ref[...], preferred_element_type=_f32)


def _mid_body(sc_ref, r0_ref, b0_ref, m1_ref, rt1_ref,
              t1_ref, r1_ref):
    v = sc_ref[...]
    fused = v[0] + v[1]
    cnt = jnp.maximum(fused[:, _DH:], 1.0)
    aggr = fused[:, :_DH] / cnt
    h1 = _elu(aggr + r0_ref[...] + b0_ref[...])
    t1_ref[...] = jnp.dot(h1, m1_ref[...],
                          preferred_element_type=_f32).astype(jnp.bfloat16)
    r1_ref[...] = jnp.dot(h1, rt1_ref[...], preferred_element_type=_f32)


def _fin_body(s1_ref, sc_ref, r1_ref, b1_ref, w_ref, cb2_ref,
              o_ref):
    v = sc_ref[...]
    cnt = jnp.maximum(v[0, :, _DH:] + v[1, :, _DH:], 1.0)
    s1v = s1_ref[...]
    aggr = (s1v[0] + s1v[1]) / cnt
    h2 = _elu(aggr + r1_ref[...] + b1_ref[...])
    o_ref[...] = (jnp.dot(h2, w_ref[...], preferred_element_type=_f32)
                  + cb2_ref[...])


def _full_spec(shape):
    return pl.BlockSpec(shape, lambda i: (0,) * len(shape))


def _row_spec(bn, w):
    return pl.BlockSpec((bn, w), lambda i: (i, 0))


_BE = 2000
_BX = 1000
_BN = 2000

_prep_call = pl.pallas_call(
    _prep_body,
    grid=(_E8 // _BE,),
    in_specs=[_row_spec(_BE, 128), _full_spec((128, 128)), _full_spec((1, 128)),
              _full_spec((128, 128)), _full_spec((1, 128)),
              _row_spec(_BX, _DIN), _full_spec((_DIN, _TW)),
              _full_spec((_DIN, _DH))],
    out_specs=[_row_spec(_BE, 128), _row_spec(_BE, 128),
               _row_spec(_BX, _TW), _row_spec(_BX, _DH)],
    out_shape=[jax.ShapeDtypeStruct((_E8, 128), _f32),
               jax.ShapeDtypeStruct((_E8, 128), _f32),
               jax.ShapeDtypeStruct((_N, _TW), jnp.bfloat16),
               jax.ShapeDtypeStruct((_N, _DH), _f32)],
)

_mid_call = pl.pallas_call(
    _mid_body,
    grid=(_N // _BN,),
    in_specs=[pl.BlockSpec((2, _BN, 2 * _DH), lambda i: (0, i, 0)),
              _row_spec(_BN, _DH), _full_spec((1, _DH)),
              _full_spec((_DH, _TW)), _full_spec((_DH, _DH))],
    out_specs=[_row_spec(_BN, _TW), _row_spec(_BN, _DH)],
    out_shape=[jax.ShapeDtypeStruct((_N, _TW), jnp.bfloat16),
               jax.ShapeDtypeStruct((_N, _DH), _f32)],
)

_fin_call = pl.pallas_call(
    _fin_body,
    grid=(_N // _BN,),
    in_specs=[pl.BlockSpec((2, _BN, _DH), lambda i: (0, i, 0)),
              pl.BlockSpec((2, _BN, 2 * _DH), lambda i: (0, i, 0)),
              _row_spec(_BN, _DH), _full_spec((1, _DH)),
              _full_spec((_DH, 10)), _full_spec((1, 10))],
    out_specs=_row_spec(_BN, 10),
    out_shape=jax.ShapeDtypeStruct((_N, 10), _f32),
)


def _build_edge_kernel(with_count):
    mesh = plsc.VectorSubcoreMesh(core_axis_name="c", subcore_axis_name="s")

    def body(*refs):
        (t_hbm, h_hbm, ei_hbm, s_out,
         src_v, dst_v, h_v, rows_v, msg_v, zer_v,
         s_sh, sem_a, sem_b, sem_c, sem_d, sem_i) = refs
        c = lax.axis_index("c")
        s = lax.axis_index("s")
        wid = s * 2 + c
        sems = (sem_a, sem_b)

        acc_w = 2 * _DH if with_count else _DH

        def zrow(i, _):
            zer_v[i, pl.ds(0, _DH)] = jnp.zeros((_DH,), _f32)
            if with_count:
                zer_v[i, pl.ds(_DH, _DH)] = jnp.zeros((_DH,), _f32)
            return 0
        lax.fori_loop(0, _RPS, zrow, 0)
        if with_count:
            def orow(i, _):
                msg_v[0, i, pl.ds(_DH, _DH)] = jnp.full((_DH,), 1.0, _f32)
                msg_v[1, i, pl.ds(_DH, _DH)] = jnp.full((_DH,), 1.0, _f32)
                return 0
            lax.fori_loop(0, _B, orow, 0)

        row0 = s * _RPS
        pltpu.sync_copy(zer_v, s_sh.at[pl.ds(row0, _RPS)])
        plsc.subcore_barrier()

        rbase = wid * (_EPW // _B)

        def chunk(gb, jv, rows_j, msg_j, sc_sem):
            @plsc.parallel_loop(0, _B, 1, unroll=4)
            def edge(e):
                hrow = h_v[gb, jv * _B + e, :]
                ab = rows_j[e, pl.ds(_KD * 2 * _DH // 2, 2 * _DH)]
                acc, _zero = plsc.unpack(
                    ab, format=plsc.PackFormat.INTERLEAVED)
                for p in range(_KD // 2):
                    abp = rows_j[e, pl.ds(2 * _DH * p, 2 * _DH)]
                    a, b = plsc.unpack(
                        abp, format=plsc.PackFormat.INTERLEAVED)
                    acc = (acc + hrow[2 * p] * a
                           + hrow[2 * p + 1] * b)
                msg_j[e, pl.ds(0, _DH)] = acc

            return pltpu.async_copy(msg_j, s_sh.at[dst_v.at[gb, jv]],
                                    sc_sem, add=True)

        def icopies(gi, buf, start):
            r0i = rbase + gi * _CPG
            e0i = r0i * _B
            op = pltpu.async_copy if start else (
                lambda s, d, m: pltpu.make_async_copy(s, d, m).wait())
            op(ei_hbm.at[0, pl.ds(r0i, _CPG)], src_v.at[buf], sem_i)
            op(ei_hbm.at[1, pl.ds(r0i, _CPG)], dst_v.at[buf], sem_i)
            op(h_hbm.at[pl.ds(e0i, _G)], h_v.at[buf], sem_i)

        icopies(0, 0, True)

        def group(g, _):
            gb = g % 2
            icopies(g, gb, False)

            @pl.when(g + 1 < _NG)
            def _prefetch():
                icopies(g + 1, (g + 1) % 2, True)

            srcs = src_v.at[gb]
            pltpu.async_copy(t_hbm.at[srcs.at[0]],
                             rows_v.at[0], sems[0]).wait()

            def pair(q, _q):
                jv0 = 2 * q
                h1 = pltpu.async_copy(t_hbm.at[srcs.at[jv0 + 1]],
                                      rows_v.at[1], sems[1])
                sca = chunk(gb, jv0, rows_v.at[0], msg_v.at[0], sem_c)
                h1.wait()
                h0 = pltpu.async_copy(t_hbm.at[srcs.at[jv0 + 2]],
                                      rows_v.at[0], sems[0])
                scb = chunk(gb, jv0 + 1, rows_v.at[1], msg_v.at[1], sem_d)
                sca.wait()
                h0.wait()
                scb.wait()
                return 0
            lax.fori_loop(0, (_CPG - 1) // 2, pair, 0)
            chunk(gb, _CPG - 1, rows_v.at[0], msg_v.at[0], sem_c).wait()
            return 0
        lax.fori_loop(0, _NG, group, 0)

        plsc.subcore_barrier()
        pltpu.sync_copy(s_sh.at[pl.ds(row0, _RPS)],
                        s_out.at[c, pl.ds(row0, _RPS)])

    acc_w = 2 * _DH if with_count else _DH
    out_type = [jax.ShapeDtypeStruct((2, _NPAD, acc_w), _f32)]
    scratch = [
        pltpu.VMEM((2, _CPG, _B), jnp.int32),
        pltpu.VMEM((2, _CPG, _B), jnp.int32),
        pltpu.VMEM((2, _G, _DH), _f32),
        pltpu.VMEM((2, _B, _TW), jnp.bfloat16),
        pltpu.VMEM((2, _B, acc_w), _f32),
        pltpu.VMEM((_RPS, acc_w), _f32),
        pltpu.VMEM_SHARED((_NPAD, acc_w), _f32),
        pltpu.SemaphoreType.DMA,
        pltpu.SemaphoreType.DMA,
        pltpu.SemaphoreType.DMA,
        pltpu.SemaphoreType.DMA,
        pltpu.SemaphoreType.DMA,
    ]

    return pl.kernel(body, out_type=out_type, mesh=mesh,
                     scratch_types=scratch,
                     compiler_params=pltpu.CompilerParams(
                         use_tc_tiling_on_sc=False,
                         needs_layout_passes=False))


_edge_call_cnt = _build_edge_kernel(with_count=True)
_edge_call_nocnt = _build_edge_kernel(with_count=False)


def kernel(x, edge_index, edge_attr, en0_W1, en0_b1, en0_W2, en0_b2, root0,
           bias0, en1_W1, en1_b1, en1_W2, en1_b2, root1, bias1, cls_W, cls_b):
    ei3 = edge_index.reshape(2, _E // _B, _B)

    eye8 = jnp.eye(8, dtype=_f32)
    k0 = jnp.kron(eye8, en0_W1)
    k1 = jnp.kron(eye8, en1_W1)
    tb0 = jnp.tile(en0_b1, 8)[None, :]
    tb1 = jnp.tile(en1_b1, 8)[None, :]
    ea8 = edge_attr.reshape(_E8, 128)
    perm = jnp.array(_BF_PERM, dtype=jnp.int32)
    m0 = en0_W2.reshape(_DE, _DIN, _DH).transpose(1, 0, 2).reshape(_DIN, _KD * _DH)
    m0aug = jnp.concatenate([m0, en0_b2.reshape(_DIN, _DH),
                             jnp.zeros((_DIN, _DH), _f32)], axis=1)[:, perm]
    he0_r, he1_r, t0, r0 = _prep_call(ea8, k0, tb0, k1, tb1, x, m0aug, root0)
    he0 = he0_r.reshape(_E, _DE)
    he1 = he1_r.reshape(_E, _DE)

    (s0c,) = _edge_call_cnt(t0, he0, ei3)

    m1 = en1_W2.reshape(_DE, _DH, _DH).transpose(1, 0, 2).reshape(_DH, _KD * _DH)
    m1aug = jnp.concatenate([m1, en1_b2.reshape(_DH, _DH),
                             jnp.zeros((_DH, _DH), _f32)], axis=1)[:, perm]
    t1, r1 = _mid_call(s0c, r0, bias0[None, :], m1aug, root1)

    (s1,) = _edge_call_nocnt(t1, he1, ei3)

    return _fin_call(s1, s0c, r1, bias1[None, :],
                     cls_W, cls_b[None, :])
